# baseline (speedup 1.0000x reference)
import functools

import jax
import jax.numpy as jnp
from jax import lax
from jax.experimental import pallas as pl
from jax.experimental.pallas import tpu as pltpu
from jax.experimental.pallas import tpu_sc as plsc

_N = 10000
_D = 128
_E = 320000
_EPS = 1e-5

_NC = 2
_NS = 16
_NT = _NC * _NS
_CH = 128
_CPT = 80
_EPT = _CPT * _CH
_EPAD = _NT * _EPT
_SLOTS = 4
_NROW = 10240
_RPT = _NROW // _NS
_ZR = 128
_NPAD = 10240
_DWPT = _NPAD // _NS


def _edge_weights(edge_attr, w0, b0, w1, b1):
    bE = 10000

    def kern(ea_ref, w0_ref, b0_ref, w1_ref, b1_ref, out_ref):
        ea = ea_ref[...]
        z0 = jnp.dot(ea, w0_ref[...], preferred_element_type=jnp.float32) + b0_ref[...]
        z1 = jnp.dot(ea, w1_ref[...], preferred_element_type=jnp.float32) + b1_ref[...]
        z = jnp.concatenate([z0, z1], axis=1)
        out_ref[...] = jnp.maximum(z, 0.0) + jnp.log1p(jnp.exp(-jnp.abs(z)))

    return pl.pallas_call(
        kern,
        grid=(_E // bE,),
        in_specs=[
            pl.BlockSpec((bE, 16), lambda i: (i, 0)),
            pl.BlockSpec((16, 1), lambda i: (0, 0)),
            pl.BlockSpec((1, 1), lambda i: (0, 0)),
            pl.BlockSpec((16, 1), lambda i: (0, 0)),
            pl.BlockSpec((1, 1), lambda i: (0, 0)),
        ],
        out_specs=pl.BlockSpec((bE, 2), lambda i: (i, 0)),
        out_shape=jax.ShapeDtypeStruct((_E, 2), jnp.float32),
    )(edge_attr, w0, b0.reshape(1, 1), w1, b1.reshape(1, 1))


def _sc_scatter(x, src_t, dst_t, w_t):
    mesh = plsc.VectorSubcoreMesh(core_axis_name="c", subcore_axis_name="s")

    @functools.partial(
        pl.kernel,
        mesh=mesh,
        out_type=(
            jax.ShapeDtypeStruct((_NC, _NROW, _D), jnp.float32),
            jax.ShapeDtypeStruct((_NC * _NPAD,), jnp.float32),
        ),
        scratch_types=[
            pltpu.VMEM((_SLOTS, 1, _CH), jnp.int32),
            pltpu.VMEM((_SLOTS, 1, _CH), jnp.int32),
            pltpu.VMEM((_SLOTS, 1, _CH), jnp.float32),
            pltpu.VMEM((2, _CH, _D), jnp.float32),
            pltpu.VMEM((_DWPT,), jnp.float32),
            pltpu.VMEM_SHARED((_NROW, _D), jnp.float32),
            pltpu.VMEM_SHARED((_NPAD,), jnp.float32),
            pltpu.SemaphoreType.DMA,
            pltpu.SemaphoreType.DMA,
            pltpu.SemaphoreType.DMA,
        ],
    )
    def k(x_hbm, src_hbm, dst_hbm, w_hbm, out_hbm, dw_hbm,
          srcb, dstb, wb, rowsb, zdw_v, acc_s, dw_s, gsem, ssem, isem):
        cid = lax.axis_index("c")
        sid = lax.axis_index("s")
        wid = cid * _NS + sid

        zero16 = jnp.zeros((16,), jnp.float32)

        def zrow(r, c):
            for j in range(_D // 16):
                rowsb[0, r, pl.ds(j * 16, 16)] = zero16
            return c
        lax.fori_loop(0, _ZR, zrow, 0)

        def zdw(i, c):
            zdw_v[pl.ds(i * 16, 16)] = zero16
            return c
        lax.fori_loop(0, _DWPT // 16, zdw, 0)

        for t in range(_RPT // _ZR):
            pltpu.sync_copy(rowsb.at[0],
                            acc_s.at[pl.ds(sid * _RPT + t * _ZR, _ZR)])
        pltpu.sync_copy(zdw_v, dw_s.at[pl.ds(sid * _DWPT, _DWPT)])
        plsc.subcore_barrier()

        pltpu.sync_copy(src_hbm.at[wid, 0], srcb.at[0])
        pltpu.sync_copy(dst_hbm.at[wid, 0], dstb.at[0])
        pltpu.sync_copy(w_hbm.at[wid, 0], wb.at[0])
        pltpu.async_copy(x_hbm.at[srcb.at[0, 0]], rowsb.at[0], gsem)
        pltpu.async_copy(src_hbm.at[wid, 1], srcb.at[1], isem)
        pltpu.async_copy(dst_hbm.at[wid, 1], dstb.at[1], isem)
        pltpu.async_copy(w_hbm.at[wid, 1], wb.at[1], isem)

        def quad(p, c):
            for b in range(4):
                i = p * 4 + b
                rs, rs1 = b % 2, (b + 1) % 2
                is0, is1, is2 = b, (b + 1) % 4, (b + 2) % 4

                pltpu.make_async_copy(
                    x_hbm.at[srcb.at[is0, 0]], rowsb.at[rs], gsem).wait()

                @pl.when(i + 1 < _CPT)
                def _():
                    pltpu.make_async_copy(
                        src_hbm.at[wid, 0], srcb.at[is1], isem).wait()
                    pltpu.make_async_copy(
                        dst_hbm.at[wid, 0], dstb.at[is1], isem).wait()
                    pltpu.make_async_copy(
                        w_hbm.at[wid, 0], wb.at[is1], isem).wait()

                    @pl.when(i >= 1)
                    def _():
                        pltpu.make_async_copy(
                            rowsb.at[rs1], acc_s.at[dstb.at[is1, 0]],
                            ssem).wait()
                    pltpu.async_copy(
                        x_hbm.at[srcb.at[is1, 0]], rowsb.at[rs1], gsem)

                def grp(g, c2):
                    wv = wb[is0, 0, pl.ds(g * 16, 16)]
                    for kk in range(16):
                        ws = wv[kk]
                        e = g * 16 + kk
                        for j in range(_D // 16):
                            sl = pl.ds(j * 16, 16)
                            rowsb[rs, e, sl] = rowsb[rs, e, sl] * ws
                    return c2
                lax.fori_loop(0, _CH // 16, grp, 0)

                pltpu.async_copy(rowsb.at[rs], acc_s.at[dstb.at[is0, 0]],
                                 ssem, add=True)
                pltpu.sync_copy(wb.at[is0, 0], dw_s.at[dstb.at[is0, 0]],
                                add=True)

                @pl.when(i + 2 < _CPT)
                def _():
                    pltpu.async_copy(src_hbm.at[wid, i + 2], srcb.at[is2],
                                     isem)
                    pltpu.async_copy(dst_hbm.at[wid, i + 2], dstb.at[is2],
                                     isem)
                    pltpu.async_copy(w_hbm.at[wid, i + 2], wb.at[is2], isem)
            return c
        lax.fori_loop(0, _CPT // 4, quad, 0)

        pltpu.make_async_copy(rowsb.at[0], acc_s.at[dstb.at[0, 0]],
                              ssem).wait()
        pltpu.make_async_copy(rowsb.at[1], acc_s.at[dstb.at[1, 0]],
                              ssem).wait()

        plsc.subcore_barrier()
        pltpu.sync_copy(acc_s.at[pl.ds(sid * _RPT, _RPT)],
                        out_hbm.at[cid, pl.ds(sid * _RPT, _RPT)])
        pltpu.sync_copy(dw_s.at[pl.ds(sid * _DWPT, _DWPT)],
                        dw_hbm.at[pl.ds(cid * _NPAD + sid * _DWPT, _DWPT)])

    return k(x, src_t, dst_t, w_t)


def _combine(p0, p1, dw0, dw1, xin, lin_W, lin_b, ln_g, ln_bt,
             fc_W=None, fc_b=None):
    bN = 1000
    final = fc_W is not None

    def kern(*refs):
        if final:
            (p0_ref, p1_ref, dw0_ref, dw1_ref, x_ref, w_ref, b_ref,
             g_ref, bt_ref, fw_ref, fb_ref, out_ref) = refs
        else:
            (p0_ref, p1_ref, dw0_ref, dw1_ref, x_ref, w_ref, b_ref,
             g_ref, bt_ref, out_ref) = refs
        dw = dw0_ref[...] + dw1_ref[...]
        aggr = p0_ref[...] + p1_ref[...] - dw * x_ref[...]
        h = lax.dot_general(aggr, w_ref[...], (((1,), (1,)), ((), ())),
                            preferred_element_type=jnp.float32) + b_ref[...]
        h = jnp.maximum(h, 0.0)
        mu = jnp.mean(h, axis=1, keepdims=True)
        hc = h - mu
        var = jnp.mean(hc * hc, axis=1, keepdims=True)
        hn = hc * lax.rsqrt(var + _EPS) * g_ref[...] + bt_ref[...]
        if final:
            hn = lax.dot_general(hn, fw_ref[...], (((1,), (1,)), ((), ())),
                                 preferred_element_type=jnp.float32) + fb_ref[...]
        out_ref[...] = hn

    row = pl.BlockSpec((bN, _D), lambda i: (i, 0))
    col = pl.BlockSpec((bN, 1), lambda i: (i, 0))
    full = pl.BlockSpec((_D, _D), lambda i: (0, 0))
    vec = pl.BlockSpec((1, _D), lambda i: (0, 0))
    in_specs = [row, row, col, col, row, full, vec, vec, vec]
    args = [p0, p1, dw0, dw1, xin, lin_W, lin_b.reshape(1, _D),
            ln_g.reshape(1, _D), ln_bt.reshape(1, _D)]
    if final:
        in_specs += [full, vec]
        args += [fc_W, fc_b.reshape(1, _D)]

    return pl.pallas_call(
        kern,
        grid=(_N // bN,),
        in_specs=in_specs,
        out_specs=row,
        out_shape=jax.ShapeDtypeStruct((_N, _D), jnp.float32),
    )(*args)


def kernel(x, edge_index, edge_attr, lin0_W, lin0_b, emlp0_W, emlp0_b,
           ln0_g, ln0_bt, lin1_W, lin1_b, emlp1_W, emlp1_b, ln1_g, ln1_bt,
           fc_W, fc_b):
    src = edge_index[0]
    dst = edge_index[1]

    w01 = _edge_weights(edge_attr, emlp0_W, emlp0_b, emlp1_W, emlp1_b)

    pad = _EPAD - _E
    src_t = jnp.pad(src, (0, pad)).reshape(_NT, _CPT, 1, _CH)
    dst_t = jnp.pad(dst, (0, pad)).reshape(_NT, _CPT, 1, _CH)
    w0_t = jnp.pad(w01[:, 0], (0, pad)).reshape(_NT, _CPT, 1, _CH)
    w1_t = jnp.pad(w01[:, 1], (0, pad)).reshape(_NT, _CPT, 1, _CH)

    p, dwp = _sc_scatter(x, src_t, dst_t, w0_t)
    dwp = dwp.reshape(_NC, _NPAD)
    dw0 = dwp[0, :_N].reshape(_N, 1)
    dw1 = dwp[1, :_N].reshape(_N, 1)
    h = _combine(p[0, :_N], p[1, :_N], dw0, dw1, x,
                 lin0_W, lin0_b, ln0_g, ln0_bt)

    p, dwp = _sc_scatter(h, src_t, dst_t, w1_t)
    dwp = dwp.reshape(_NC, _NPAD)
    dw0 = dwp[0, :_N].reshape(_N, 1)
    dw1 = dwp[1, :_N].reshape(_N, 1)
    return _combine(p[0, :_N], p[1, :_N], dw0, dw1, h,
                    lin1_W, lin1_b, ln1_g, ln1_bt, fc_W, fc_b)
